# Initial kernel scaffold; baseline (speedup 1.0000x reference)
import jax
import jax.numpy as jnp
from jax.experimental import pallas as pl


def kernel(x, w1, g1, b1, wr, br, we, ge, be, w3, g3, b3):
    raise NotImplementedError("write your pallas kernel here")


# speedup vs baseline: 1.8337x; 1.8337x over previous
import jax
import jax.numpy as jnp
from jax.experimental import pallas as pl

_E = 4
_EPS = 1e-5
_TN = 512


def _stage_a(x_ref, w1_ref, g1_ref, b1_ref, wr_ref, br_ref,
             y_ref, tw_ref, ti_ref):
    xb = x_ref[...]
    y = jnp.dot(xb, w1_ref[...], preferred_element_type=jnp.float32)
    y = y * g1_ref[...] + b1_ref[...]
    y = y * jax.nn.sigmoid(y)
    y_ref[...] = y
    lg = jnp.dot(y, wr_ref[...], preferred_element_type=jnp.float32)
    lg = lg + br_ref[...]
    m = jnp.max(lg, axis=1, keepdims=True)
    ex = jnp.exp(lg - m)
    p = ex / jnp.sum(ex, axis=1, keepdims=True)
    p0 = p[:, 0:1]
    p1 = p[:, 1:2]
    p2 = p[:, 2:3]
    p3 = p[:, 3:4]
    m1 = jnp.maximum(jnp.maximum(p0, p1), jnp.maximum(p2, p3))
    i1 = jnp.where(p0 == m1, 0,
                   jnp.where(p1 == m1, 1,
                             jnp.where(p2 == m1, 2, 3))).astype(jnp.int32)
    q0 = jnp.where(i1 == 0, -1.0, p0)
    q1 = jnp.where(i1 == 1, -1.0, p1)
    q2 = jnp.where(i1 == 2, -1.0, p2)
    q3 = jnp.where(i1 == 3, -1.0, p3)
    m2 = jnp.maximum(jnp.maximum(q0, q1), jnp.maximum(q2, q3))
    i2 = jnp.where(q0 == m2, 0,
                   jnp.where(q1 == m2, 1,
                             jnp.where(q2 == m2, 2, 3))).astype(jnp.int32)
    s = m1 + m2
    tw_ref[:, 0:1] = m1 / s
    tw_ref[:, 1:2] = m2 / s
    ti_ref[:, 0:1] = i1
    ti_ref[:, 1:2] = i2


def _stage_b(r_ref, w_ref, g_ref, b_ref, o_ref):
    r = r_ref[0]
    eo = jnp.dot(r, w_ref[0], preferred_element_type=jnp.float32)
    eo = eo * g_ref[0] + b_ref[0]
    eo = eo * jax.nn.sigmoid(eo)
    o_ref[0] = eo


def _stage_c(m1_ref, m2_ref, tw_ref, w3_ref, g3_ref, b3_ref, xr_ref, o_ref):
    m = (m1_ref[...] * tw_ref[:, 0:1] + m2_ref[...] * tw_ref[:, 1:2])
    y = jnp.dot(m, w3_ref[...], preferred_element_type=jnp.float32)
    y = y * g3_ref[...] + b3_ref[...]
    y = y * jax.nn.sigmoid(y)
    o_ref[...] = y + xr_ref[...]


def kernel(x, w1, g1, b1, wr, br, we, ge, be, w3, g3, b3):
    B, C1, H, W = x.shape
    N = H * W
    C = w1.shape[0]
    C2 = w3.shape[0]
    inv = 1.0 / (1.0 + _EPS) ** 0.5

    x2 = x.reshape(C1, N)
    xt = x2.T
    w1m = w1[:, :, 0, 0].T
    g1s = (g1 * inv).reshape(1, C)
    b1r = b1.reshape(1, C)
    wrm = wr[:, :, 0, 0].T
    brr = br.reshape(1, _E)

    grid = N // _TN
    y, tw, ti = pl.pallas_call(
        _stage_a,
        grid=(grid,),
        in_specs=[
            pl.BlockSpec((_TN, C1), lambda i: (i, 0)),
            pl.BlockSpec((C1, C), lambda i: (0, 0)),
            pl.BlockSpec((1, C), lambda i: (0, 0)),
            pl.BlockSpec((1, C), lambda i: (0, 0)),
            pl.BlockSpec((C, _E), lambda i: (0, 0)),
            pl.BlockSpec((1, _E), lambda i: (0, 0)),
        ],
        out_specs=[
            pl.BlockSpec((_TN, C), lambda i: (i, 0)),
            pl.BlockSpec((_TN, 2), lambda i: (i, 0)),
            pl.BlockSpec((_TN, 2), lambda i: (i, 0)),
        ],
        out_shape=[
            jax.ShapeDtypeStruct((N, C), jnp.float32),
            jax.ShapeDtypeStruct((N, 2), jnp.float32),
            jax.ShapeDtypeStruct((N, 2), jnp.int32),
        ],
    )(xt, w1m, g1s, b1r, wrm, brr)

    i1 = ti[:, 0]
    i2 = ti[:, 1]
    tw1 = tw[:, 0]
    tw2 = tw[:, 1]
    yT = y.T
    rows = jnp.arange(N)
    flat = jnp.arange(N * C)

    r_list, rank_list = [], []
    for ei in range(_E):
        mask = (i1 == ei) | (i2 == ei)
        S_safe = jnp.maximum(jnp.sum(mask.astype(jnp.int32)), 1)
        sel = jnp.nonzero(mask, size=N, fill_value=0)[0]
        cidx = jnp.minimum(flat // S_safe, C - 1)
        tok = sel[flat % S_safe]
        r_list.append(yT[cidx, tok].reshape(N, C))
        rank_list.append(jnp.cumsum(mask.astype(jnp.int32)) - 1)

    r_all = jnp.stack(r_list)
    rank_all = jnp.stack(rank_list)

    wt = jnp.transpose(we[:, :, :, 1, 1], (0, 2, 1))
    ges = (ge * inv).reshape(_E, 1, C)
    ber = be.reshape(_E, 1, C)

    contrib = pl.pallas_call(
        _stage_b,
        grid=(_E, grid),
        in_specs=[
            pl.BlockSpec((1, _TN, C), lambda e, i: (e, i, 0)),
            pl.BlockSpec((1, C, C), lambda e, i: (e, 0, 0)),
            pl.BlockSpec((1, 1, C), lambda e, i: (e, 0, 0)),
            pl.BlockSpec((1, 1, C), lambda e, i: (e, 0, 0)),
        ],
        out_specs=pl.BlockSpec((1, _TN, C), lambda e, i: (e, i, 0)),
        out_shape=jax.ShapeDtypeStruct((_E, N, C), jnp.float32),
    )(r_all, wt, ges, ber)

    r1 = rank_all[i1, rows]
    r2 = rank_all[i2, rows]
    cf = contrib.reshape(_E * N, C)
    cf1 = cf[i1 * N + r1]
    cf2 = cf[i2 * N + r2]

    w3m = w3[:, :, 0, 0].T
    g3s = (g3 * inv).reshape(1, C2)
    b3r = b3.reshape(1, C2)
    out = pl.pallas_call(
        _stage_c,
        grid=(grid,),
        in_specs=[
            pl.BlockSpec((_TN, C), lambda i: (i, 0)),
            pl.BlockSpec((_TN, C), lambda i: (i, 0)),
            pl.BlockSpec((_TN, 2), lambda i: (i, 0)),
            pl.BlockSpec((C, C2), lambda i: (0, 0)),
            pl.BlockSpec((1, C2), lambda i: (0, 0)),
            pl.BlockSpec((1, C2), lambda i: (0, 0)),
            pl.BlockSpec((_TN, C2), lambda i: (i, 0)),
        ],
        out_specs=pl.BlockSpec((_TN, C2), lambda i: (i, 0)),
        out_shape=jax.ShapeDtypeStruct((N, C2), jnp.float32),
    )(cf1, cf2, tw, w3m, g3s, b3r, xt)

    return out.T.reshape(B, C2, H, W)


# speedup vs baseline: 1.8352x; 1.0008x over previous
import jax
import jax.numpy as jnp
from jax.experimental import pallas as pl

_E = 4
_EPS = 1e-5
_TN = 2048


def _stage_a(x_ref, w1_ref, g1_ref, b1_ref, wr_ref, br_ref,
             y_ref, tw_ref, ti_ref):
    xb = x_ref[...]
    y = jnp.dot(xb, w1_ref[...], preferred_element_type=jnp.float32)
    y = y * g1_ref[...] + b1_ref[...]
    y = y * jax.nn.sigmoid(y)
    y_ref[...] = y
    lg = jnp.dot(y, wr_ref[...], preferred_element_type=jnp.float32)
    lg = lg + br_ref[...]
    m = jnp.max(lg, axis=1, keepdims=True)
    ex = jnp.exp(lg - m)
    p = ex / jnp.sum(ex, axis=1, keepdims=True)
    p0 = p[:, 0:1]
    p1 = p[:, 1:2]
    p2 = p[:, 2:3]
    p3 = p[:, 3:4]
    m1 = jnp.maximum(jnp.maximum(p0, p1), jnp.maximum(p2, p3))
    i1 = jnp.where(p0 == m1, 0,
                   jnp.where(p1 == m1, 1,
                             jnp.where(p2 == m1, 2, 3))).astype(jnp.int32)
    q0 = jnp.where(i1 == 0, -1.0, p0)
    q1 = jnp.where(i1 == 1, -1.0, p1)
    q2 = jnp.where(i1 == 2, -1.0, p2)
    q3 = jnp.where(i1 == 3, -1.0, p3)
    m2 = jnp.maximum(jnp.maximum(q0, q1), jnp.maximum(q2, q3))
    i2 = jnp.where(q0 == m2, 0,
                   jnp.where(q1 == m2, 1,
                             jnp.where(q2 == m2, 2, 3))).astype(jnp.int32)
    s = m1 + m2
    tw_ref[:, 0:1] = m1 / s
    tw_ref[:, 1:2] = m2 / s
    ti_ref[:, 0:1] = i1
    ti_ref[:, 1:2] = i2


def _stage_b(r_ref, w_ref, g_ref, b_ref, o_ref):
    r = r_ref[0]
    eo = jnp.dot(r, w_ref[0], preferred_element_type=jnp.float32)
    eo = eo * g_ref[0] + b_ref[0]
    eo = eo * jax.nn.sigmoid(eo)
    o_ref[0] = eo


def _stage_c(m1_ref, m2_ref, tw_ref, w3_ref, g3_ref, b3_ref, xr_ref, o_ref):
    m = (m1_ref[...] * tw_ref[:, 0:1] + m2_ref[...] * tw_ref[:, 1:2])
    y = jnp.dot(m, w3_ref[...], preferred_element_type=jnp.float32)
    y = y * g3_ref[...] + b3_ref[...]
    y = y * jax.nn.sigmoid(y)
    o_ref[...] = y + xr_ref[...]


def kernel(x, w1, g1, b1, wr, br, we, ge, be, w3, g3, b3):
    B, C1, H, W = x.shape
    N = H * W
    C = w1.shape[0]
    C2 = w3.shape[0]
    inv = 1.0 / (1.0 + _EPS) ** 0.5

    x2 = x.reshape(C1, N)
    xt = x2.T
    w1m = w1[:, :, 0, 0].T
    g1s = (g1 * inv).reshape(1, C)
    b1r = b1.reshape(1, C)
    wrm = wr[:, :, 0, 0].T
    brr = br.reshape(1, _E)

    grid = N // _TN
    y, tw, ti = pl.pallas_call(
        _stage_a,
        grid=(grid,),
        in_specs=[
            pl.BlockSpec((_TN, C1), lambda i: (i, 0)),
            pl.BlockSpec((C1, C), lambda i: (0, 0)),
            pl.BlockSpec((1, C), lambda i: (0, 0)),
            pl.BlockSpec((1, C), lambda i: (0, 0)),
            pl.BlockSpec((C, _E), lambda i: (0, 0)),
            pl.BlockSpec((1, _E), lambda i: (0, 0)),
        ],
        out_specs=[
            pl.BlockSpec((_TN, C), lambda i: (i, 0)),
            pl.BlockSpec((_TN, 2), lambda i: (i, 0)),
            pl.BlockSpec((_TN, 2), lambda i: (i, 0)),
        ],
        out_shape=[
            jax.ShapeDtypeStruct((N, C), jnp.float32),
            jax.ShapeDtypeStruct((N, 2), jnp.float32),
            jax.ShapeDtypeStruct((N, 2), jnp.int32),
        ],
    )(xt, w1m, g1s, b1r, wrm, brr)

    i1 = ti[:, 0]
    i2 = ti[:, 1]
    tw1 = tw[:, 0]
    tw2 = tw[:, 1]
    yT = y.T
    rows = jnp.arange(N)
    flat = jnp.arange(N * C)

    r_list, rank_list = [], []
    for ei in range(_E):
        mask = (i1 == ei) | (i2 == ei)
        S_safe = jnp.maximum(jnp.sum(mask.astype(jnp.int32)), 1)
        sel = jnp.nonzero(mask, size=N, fill_value=0)[0]
        cidx = jnp.minimum(flat // S_safe, C - 1)
        tok = sel[flat % S_safe]
        r_list.append(yT[cidx, tok].reshape(N, C))
        rank_list.append(jnp.cumsum(mask.astype(jnp.int32)) - 1)

    r_all = jnp.stack(r_list)
    rank_all = jnp.stack(rank_list)

    wt = jnp.transpose(we[:, :, :, 1, 1], (0, 2, 1))
    ges = (ge * inv).reshape(_E, 1, C)
    ber = be.reshape(_E, 1, C)

    contrib = pl.pallas_call(
        _stage_b,
        grid=(_E, grid),
        in_specs=[
            pl.BlockSpec((1, _TN, C), lambda e, i: (e, i, 0)),
            pl.BlockSpec((1, C, C), lambda e, i: (e, 0, 0)),
            pl.BlockSpec((1, 1, C), lambda e, i: (e, 0, 0)),
            pl.BlockSpec((1, 1, C), lambda e, i: (e, 0, 0)),
        ],
        out_specs=pl.BlockSpec((1, _TN, C), lambda e, i: (e, i, 0)),
        out_shape=jax.ShapeDtypeStruct((_E, N, C), jnp.float32),
    )(r_all, wt, ges, ber)

    r1 = rank_all[i1, rows]
    r2 = rank_all[i2, rows]
    cf = contrib.reshape(_E * N, C)
    cf1 = cf[i1 * N + r1]
    cf2 = cf[i2 * N + r2]

    w3m = w3[:, :, 0, 0].T
    g3s = (g3 * inv).reshape(1, C2)
    b3r = b3.reshape(1, C2)
    out = pl.pallas_call(
        _stage_c,
        grid=(grid,),
        in_specs=[
            pl.BlockSpec((_TN, C), lambda i: (i, 0)),
            pl.BlockSpec((_TN, C), lambda i: (i, 0)),
            pl.BlockSpec((_TN, 2), lambda i: (i, 0)),
            pl.BlockSpec((C, C2), lambda i: (0, 0)),
            pl.BlockSpec((1, C2), lambda i: (0, 0)),
            pl.BlockSpec((1, C2), lambda i: (0, 0)),
            pl.BlockSpec((_TN, C2), lambda i: (i, 0)),
        ],
        out_specs=pl.BlockSpec((_TN, C2), lambda i: (i, 0)),
        out_shape=jax.ShapeDtypeStruct((N, C2), jnp.float32),
    )(cf1, cf2, tw, w3m, g3s, b3r, xt)

    return out.T.reshape(B, C2, H, W)
